# baseline (speedup 1.0000x reference)
import functools

import jax
import jax.numpy as jnp
from jax import lax
from jax.experimental import pallas as pl
from jax.experimental.pallas import tpu as pltpu
from jax.experimental.pallas import tpu_sc as plsc

N = 10000
E = 320000
D = 128
HID = 128
G = 64
C = 16

HALF = 80
ONES = D - HALF
NC = 2
NS = 16
BLK = 80
NBLKW = E // NS // BLK
L = 16
RPW = N // NS
NSLOT = 4

_HIGH = lax.Precision.HIGHEST


def _bcast_lane(vec, r):
    idx = jnp.full((L, 1), r, dtype=jnp.int32)
    return lax.gather(
        vec, idx,
        lax.GatherDimensionNumbers(
            offset_dims=(), collapsed_slice_dims=(0,), start_index_map=(0,)),
        slice_sizes=(1,),
        mode=lax.GatherScatterMode.PROMISE_IN_BOUNDS)


def _dense_body(with_prev, with_relu, *refs):
    if with_prev:
        (acc_ref, bias_ref, w_ref, att2_ref,
         g_ref, asrc_ref, adst_ref) = refs
        a0 = acc_ref[0]
        a1 = acc_ref[1]
        den = jnp.maximum(a1[:, ONES:ONES + 1], 1e-16)
        f = jnp.concatenate([a0, a1[:, :ONES]], axis=1) / den + bias_ref[...]
        if with_relu:
            f = jnp.maximum(f, 0.0)
    else:
        (x_ref, w_ref, att2_ref, g_ref, asrc_ref, adst_ref) = refs
        f = x_ref[...]
    h = lax.dot_general(f, w_ref[...], (((1,), (1,)), ((), ())),
                        precision=_HIGH, preferred_element_type=jnp.float32)
    g_ref[0] = h[:, :HALF]
    col = lax.broadcasted_iota(jnp.int32, (h.shape[0], HALF - ONES), 1)
    g_ref[1] = jnp.concatenate(
        [h[:, HALF:], jnp.where(col == 0, 1.0, 0.0)], axis=1)
    ab = lax.dot_general(h, att2_ref[...], (((1,), (0,)), ((), ())),
                         precision=_HIGH, preferred_element_type=jnp.float32)
    asrc_ref[...] = ab[:, 0:1]
    adst_ref[...] = ab[:, 1:2]


def _dense_stage(f_or_acc, w, att_src, att_dst, bias=None, with_relu=False):
    att2 = jnp.stack([att_src, att_dst], axis=1)
    out_shape = [
        jax.ShapeDtypeStruct((NC, N, HALF), jnp.float32),
        jax.ShapeDtypeStruct((N, 1), jnp.float32),
        jax.ShapeDtypeStruct((N, 1), jnp.float32),
    ]
    if bias is None:
        body = functools.partial(_dense_body, False, False)
        args = (f_or_acc, w, att2)
    else:
        body = functools.partial(_dense_body, True, with_relu)
        args = (f_or_acc, bias.reshape(1, HID), w, att2)
    g, asrc, adst = pl.pallas_call(body, out_shape=out_shape)(*args)
    return g, asrc.reshape(N), adst.reshape(N)


MAIN = NBLKW - 2


def _scale_rows(rows, ib, asv, adv, mvec, slot):
    @pl.loop(0, BLK // L)
    def _(j):
        si = ib[slot, 0, pl.ds(j * L, L)]
        di = ib[slot, 1, pl.ds(j * L, L)]
        a_s = plsc.load_gather(asv, [si])
        a_d = plsc.load_gather(adv, [di])
        z = a_s + a_d
        e = jnp.maximum(z, 0.2 * z)
        zb = a_d + mvec
        mb = jnp.maximum(zb, 0.2 * zb)
        ex = jnp.exp(e - mb)
        for r in range(L):
            bv = _bcast_lane(ex, r)
            row = j * L + r
            for k in range(HALF // L):
                sl = (slot, row, pl.ds(k * L, L))
                rows[sl] = rows[sl] * bv


def _edge_body(g_hbm, asrc_hbm, adst_hbm, mvec_hbm, epk_hbm, zero_hbm,
               acc_hbm, asv, adv, ib, rows, mv, acc_sh, isem, gsem):
    c = lax.axis_index("c")
    s = lax.axis_index("s")

    pltpu.sync_copy(asrc_hbm, asv)
    pltpu.sync_copy(adst_hbm, adv)
    pltpu.sync_copy(mvec_hbm, mv)

    pltpu.sync_copy(zero_hbm, acc_sh.at[pl.ds(s * RPW, RPW), :])
    plsc.subcore_barrier()

    ghalf = g_hbm.at[c]
    b0 = s * NBLKW
    mvec = mv[...]

    def fetch_idx(b, slot):
        pltpu.async_copy(epk_hbm.at[b0 + b], ib.at[slot], isem.at[slot])

    def wait_idx(b, slot):
        pltpu.make_async_copy(epk_hbm.at[b0 + b], ib.at[slot],
                              isem.at[slot]).wait()

    def start_gather(slot):
        pltpu.async_copy(ghalf.at[ib.at[slot, 0]], rows.at[slot],
                         gsem.at[slot])

    def wait_gather(slot):
        pltpu.make_async_copy(ghalf.at[ib.at[slot, 0]], rows.at[slot],
                              gsem.at[slot]).wait()

    def scatter_sync(slot):
        pltpu.sync_copy(rows.at[slot], acc_sh.at[ib.at[slot, 1]], add=True)

    for k in range(NSLOT):
        fetch_idx(k, k)
    for k in range(2):
        wait_idx(k, k)
        start_gather(k)

    @pl.loop(0, MAIN, step=NSLOT)
    def _(p):
        for k in range(NSLOT):
            b = p + k
            wait_gather(k)
            _scale_rows(rows, ib, asv, adv, mvec, k)
            scatter_sync(k)

            @pl.when(b + NSLOT < NBLKW)
            def _():
                fetch_idx(b + NSLOT, k)

            wait_idx(b + 2, (k + 2) % NSLOT)
            start_gather((k + 2) % NSLOT)

    for t in range(2):
        wait_gather(t)
        _scale_rows(rows, ib, asv, adv, mvec, t)
        scatter_sync(t)

    plsc.subcore_barrier()
    pltpu.sync_copy(acc_sh.at[pl.ds(s * RPW, RPW), :],
                    acc_hbm.at[c, pl.ds(s * RPW, RPW), :])


_edge_kernel = pl.kernel(
    _edge_body,
    out_type=jax.ShapeDtypeStruct((NC, N, HALF), jnp.float32),
    mesh=plsc.VectorSubcoreMesh(core_axis_name="c", subcore_axis_name="s"),
    scratch_types=[
        pltpu.VMEM((N,), jnp.float32),
        pltpu.VMEM((N,), jnp.float32),
        pltpu.VMEM((NSLOT, 2, BLK), jnp.int32),
        pltpu.VMEM((NSLOT, BLK, HALF), jnp.float32),
        pltpu.VMEM((L,), jnp.float32),
        pltpu.VMEM_SHARED((N, HALF), jnp.float32),
        pltpu.SemaphoreType.DMA((NSLOT,)),
        pltpu.SemaphoreType.DMA((NSLOT,)),
    ],
    compiler_params=pltpu.CompilerParams(use_tc_tiling_on_sc=False,
                                         needs_layout_passes=False),
)


def _edge_stage(g, asrc, adst, epk, zero_rows):
    m = jnp.max(asrc)
    mvec = jnp.full((L,), m, dtype=jnp.float32)
    return _edge_kernel(g, asrc, adst, mvec, epk, zero_rows)


def _head_body(acc_ref, bias_ref, batch_ref, wlin_ref, blin_ref,
               out_ref, xmid_ref):
    a0 = acc_ref[0]
    a1 = acc_ref[1]
    den = jnp.maximum(a1[:, ONES:ONES + 1], 1e-16)
    f = jnp.concatenate([a0, a1[:, :ONES]], axis=1) / den
    bt = batch_ref[...]
    gids = lax.broadcasted_iota(jnp.int32, (G, N), 0)
    oh = jnp.where(gids == bt, 1.0, 0.0)
    psum = lax.dot_general(oh, f, (((1,), (0,)), ((), ())),
                           precision=_HIGH, preferred_element_type=jnp.float32)
    cnt = jnp.sum(oh, axis=1, keepdims=True)
    pooled = psum / jnp.maximum(cnt, 1.0) + bias_ref[...]
    xmid_ref[...] = pooled
    out_ref[...] = lax.dot_general(
        pooled, wlin_ref[...], (((1,), (1,)), ((), ())),
        precision=_HIGH, preferred_element_type=jnp.float32) + blin_ref[...]


def _head_stage(acc, b3, batch, wlin, blin):
    out_shape = [
        jax.ShapeDtypeStruct((G, C), jnp.float32),
        jax.ShapeDtypeStruct((G, HID), jnp.float32),
    ]
    return pl.pallas_call(_head_body, out_shape=out_shape)(
        acc, b3.reshape(1, HID), batch.reshape(1, N).astype(jnp.int32),
        wlin, blin.reshape(1, C))


def kernel(x, edge_index, batch, W1, att_src1, att_dst1, b1,
           W2, att_src2, att_dst2, b2, W3, att_src3, att_dst3, b3,
           Wlin, blin):
    epk = jnp.transpose(edge_index.astype(jnp.int32).reshape(2, E // BLK, BLK),
                        (1, 0, 2))
    zero_rows = jnp.zeros((RPW, HALF), jnp.float32)

    g1, as1, ad1 = _dense_stage(x, W1, att_src1, att_dst1)
    acc1 = _edge_stage(g1, as1, ad1, epk, zero_rows)

    g2, as2, ad2 = _dense_stage(acc1, W2, att_src2, att_dst2,
                                bias=b1, with_relu=True)
    acc2 = _edge_stage(g2, as2, ad2, epk, zero_rows)

    g3, as3, ad3 = _dense_stage(acc2, W3, att_src3, att_dst3,
                                bias=b2, with_relu=True)
    acc3 = _edge_stage(g3, as3, ad3, epk, zero_rows)

    return _head_stage(acc3, b3, batch, Wlin, blin)


# speedup vs baseline: 1.5238x; 1.5238x over previous
import functools

import jax
import jax.numpy as jnp
from jax import lax
from jax.experimental import pallas as pl
from jax.experimental.pallas import tpu as pltpu
from jax.experimental.pallas import tpu_sc as plsc

N = 10000
E = 320000
D = 128
HID = 128
G = 64
C = 16

HALF = 80
ONES = D - HALF
NC = 2
NS = 16
BLK = 80
NBLKW = E // NS // BLK
L = 16
RPW = N // NS
NSLOT = 4

_HIGH = lax.Precision.HIGHEST


def _bcast_lane(vec, r):
    idx = jnp.full((L, 1), r, dtype=jnp.int32)
    return lax.gather(
        vec, idx,
        lax.GatherDimensionNumbers(
            offset_dims=(), collapsed_slice_dims=(0,), start_index_map=(0,)),
        slice_sizes=(1,),
        mode=lax.GatherScatterMode.PROMISE_IN_BOUNDS)


def _dense_body(with_prev, with_relu, *refs):
    if with_prev:
        (acc_ref, bias_ref, w_ref, att2_ref,
         g_ref, asrc_ref, adst_ref) = refs
        a0 = acc_ref[0]
        a1 = acc_ref[1]
        den = jnp.maximum(a1[:, ONES:ONES + 1], 1e-16)
        f = jnp.concatenate([a0, a1[:, :ONES]], axis=1) / den + bias_ref[...]
        if with_relu:
            f = jnp.maximum(f, 0.0)
    else:
        (x_ref, w_ref, att2_ref, g_ref, asrc_ref, adst_ref) = refs
        f = x_ref[...]
    h = lax.dot_general(f, w_ref[...], (((1,), (1,)), ((), ())),
                        precision=_HIGH, preferred_element_type=jnp.float32)
    g_ref[0] = h[:, :HALF]
    col = lax.broadcasted_iota(jnp.int32, (h.shape[0], HALF - ONES), 1)
    g_ref[1] = jnp.concatenate(
        [h[:, HALF:], jnp.where(col == 0, 1.0, 0.0)], axis=1)
    ab = lax.dot_general(h, att2_ref[...], (((1,), (0,)), ((), ())),
                         precision=_HIGH, preferred_element_type=jnp.float32)
    asrc_ref[...] = ab[:, 0:1]
    adst_ref[...] = ab[:, 1:2]


def _dense_stage(f_or_acc, w, att_src, att_dst, bias=None, with_relu=False):
    att2 = jnp.stack([att_src, att_dst], axis=1)
    out_shape = [
        jax.ShapeDtypeStruct((NC, N, HALF), jnp.float32),
        jax.ShapeDtypeStruct((N, 1), jnp.float32),
        jax.ShapeDtypeStruct((N, 1), jnp.float32),
    ]
    if bias is None:
        body = functools.partial(_dense_body, False, False)
        args = (f_or_acc, w, att2)
    else:
        body = functools.partial(_dense_body, True, with_relu)
        args = (f_or_acc, bias.reshape(1, HID), w, att2)
    g, asrc, adst = pl.pallas_call(body, out_shape=out_shape)(*args)
    return g, asrc.reshape(N), adst.reshape(N)


MAIN = NBLKW - 2


def _scale_rows(rows, ib, asv, adv, mvec, slot):
    @plsc.parallel_loop(0, BLK // L, unroll=BLK // L)
    def _(j):
        si = ib[slot, 0, pl.ds(j * L, L)]
        di = ib[slot, 1, pl.ds(j * L, L)]
        a_s = plsc.load_gather(asv, [si])
        a_d = plsc.load_gather(adv, [di])
        z = a_s + a_d
        e = jnp.maximum(z, 0.2 * z)
        zb = a_d + mvec
        mb = jnp.maximum(zb, 0.2 * zb)
        ex = jnp.exp(e - mb)
        for r in range(L):
            bv = _bcast_lane(ex, r)
            row = j * L + r
            for k in range(HALF // L):
                sl = (slot, row, pl.ds(k * L, L))
                rows[sl] = rows[sl] * bv


def _edge_body(g_hbm, asrc_hbm, adst_hbm, mvec_hbm, epk_hbm, zero_hbm,
               acc_hbm, asv, adv, ib, rows, mv, acc_sh, isem, gsem):
    c = lax.axis_index("c")
    s = lax.axis_index("s")

    pltpu.sync_copy(asrc_hbm, asv)
    pltpu.sync_copy(adst_hbm, adv)
    pltpu.sync_copy(mvec_hbm, mv)

    pltpu.sync_copy(zero_hbm, acc_sh.at[pl.ds(s * RPW, RPW), :])
    plsc.subcore_barrier()

    ghalf = g_hbm.at[c]
    b0 = s * NBLKW
    mvec = mv[...]

    def fetch_idx(b, slot):
        pltpu.async_copy(epk_hbm.at[b0 + b], ib.at[slot], isem.at[slot])

    def wait_idx(b, slot):
        pltpu.make_async_copy(epk_hbm.at[b0 + b], ib.at[slot],
                              isem.at[slot]).wait()

    def start_gather(slot):
        pltpu.async_copy(ghalf.at[ib.at[slot, 0]], rows.at[slot],
                         gsem.at[slot])

    def wait_gather(slot):
        pltpu.make_async_copy(ghalf.at[ib.at[slot, 0]], rows.at[slot],
                              gsem.at[slot]).wait()

    def scatter_sync(slot):
        pltpu.sync_copy(rows.at[slot], acc_sh.at[ib.at[slot, 1]], add=True)

    for k in range(NSLOT):
        fetch_idx(k, k)
    for k in range(2):
        wait_idx(k, k)
        start_gather(k)

    @pl.loop(0, MAIN, step=NSLOT)
    def _(p):
        for k in range(NSLOT):
            b = p + k
            wait_gather(k)
            _scale_rows(rows, ib, asv, adv, mvec, k)
            scatter_sync(k)

            @pl.when(b + NSLOT < NBLKW)
            def _():
                fetch_idx(b + NSLOT, k)

            wait_idx(b + 2, (k + 2) % NSLOT)
            start_gather((k + 2) % NSLOT)

    for t in range(2):
        wait_gather(t)
        _scale_rows(rows, ib, asv, adv, mvec, t)
        scatter_sync(t)

    plsc.subcore_barrier()
    pltpu.sync_copy(acc_sh.at[pl.ds(s * RPW, RPW), :],
                    acc_hbm.at[c, pl.ds(s * RPW, RPW), :])


_edge_kernel = pl.kernel(
    _edge_body,
    out_type=jax.ShapeDtypeStruct((NC, N, HALF), jnp.float32),
    mesh=plsc.VectorSubcoreMesh(core_axis_name="c", subcore_axis_name="s"),
    scratch_types=[
        pltpu.VMEM((N,), jnp.float32),
        pltpu.VMEM((N,), jnp.float32),
        pltpu.VMEM((NSLOT, 2, BLK), jnp.int32),
        pltpu.VMEM((NSLOT, BLK, HALF), jnp.float32),
        pltpu.VMEM((L,), jnp.float32),
        pltpu.VMEM_SHARED((N, HALF), jnp.float32),
        pltpu.SemaphoreType.DMA((NSLOT,)),
        pltpu.SemaphoreType.DMA((NSLOT,)),
    ],
    compiler_params=pltpu.CompilerParams(use_tc_tiling_on_sc=False,
                                         needs_layout_passes=False),
)


def _edge_stage(g, asrc, adst, epk, zero_rows):
    m = jnp.max(asrc)
    mvec = jnp.full((L,), m, dtype=jnp.float32)
    return _edge_kernel(g, asrc, adst, mvec, epk, zero_rows)


def _head_body(acc_ref, bias_ref, batch_ref, wlin_ref, blin_ref,
               out_ref, xmid_ref):
    a0 = acc_ref[0]
    a1 = acc_ref[1]
    den = jnp.maximum(a1[:, ONES:ONES + 1], 1e-16)
    f = jnp.concatenate([a0, a1[:, :ONES]], axis=1) / den
    bt = batch_ref[...]
    gids = lax.broadcasted_iota(jnp.int32, (G, N), 0)
    oh = jnp.where(gids == bt, 1.0, 0.0)
    psum = lax.dot_general(oh, f, (((1,), (0,)), ((), ())),
                           precision=_HIGH, preferred_element_type=jnp.float32)
    cnt = jnp.sum(oh, axis=1, keepdims=True)
    pooled = psum / jnp.maximum(cnt, 1.0) + bias_ref[...]
    xmid_ref[...] = pooled
    out_ref[...] = lax.dot_general(
        pooled, wlin_ref[...], (((1,), (1,)), ((), ())),
        precision=_HIGH, preferred_element_type=jnp.float32) + blin_ref[...]


def _head_stage(acc, b3, batch, wlin, blin):
    out_shape = [
        jax.ShapeDtypeStruct((G, C), jnp.float32),
        jax.ShapeDtypeStruct((G, HID), jnp.float32),
    ]
    return pl.pallas_call(_head_body, out_shape=out_shape)(
        acc, b3.reshape(1, HID), batch.reshape(1, N).astype(jnp.int32),
        wlin, blin.reshape(1, C))


def kernel(x, edge_index, batch, W1, att_src1, att_dst1, b1,
           W2, att_src2, att_dst2, b2, W3, att_src3, att_dst3, b3,
           Wlin, blin):
    epk = jnp.transpose(edge_index.astype(jnp.int32).reshape(2, E // BLK, BLK),
                        (1, 0, 2))
    zero_rows = jnp.zeros((RPW, HALF), jnp.float32)

    g1, as1, ad1 = _dense_stage(x, W1, att_src1, att_dst1)
    acc1 = _edge_stage(g1, as1, ad1, epk, zero_rows)

    g2, as2, ad2 = _dense_stage(acc1, W2, att_src2, att_dst2,
                                bias=b1, with_relu=True)
    acc2 = _edge_stage(g2, as2, ad2, epk, zero_rows)

    g3, as3, ad3 = _dense_stage(acc2, W3, att_src3, att_dst3,
                                bias=b2, with_relu=True)
    acc3 = _edge_stage(g3, as3, ad3, epk, zero_rows)

    return _head_stage(acc3, b3, batch, Wlin, blin)


# speedup vs baseline: 1.7441x; 1.1445x over previous
import functools

import jax
import jax.numpy as jnp
from jax import lax
from jax.experimental import pallas as pl
from jax.experimental.pallas import tpu as pltpu
from jax.experimental.pallas import tpu_sc as plsc

N = 10000
E = 320000
D = 128
HID = 128
G = 64
C = 16

HALF = 80
ONES = D - HALF
NC = 2
NS = 16
BLK = 80
NBLKW = E // NS // BLK
L = 16
RPW = N // NS
NSLOT = 4

_HIGH = lax.Precision.HIGHEST


def _bcast_lane(vec, r):
    idx = jnp.full((L, 1), r, dtype=jnp.int32)
    return lax.gather(
        vec, idx,
        lax.GatherDimensionNumbers(
            offset_dims=(), collapsed_slice_dims=(0,), start_index_map=(0,)),
        slice_sizes=(1,),
        mode=lax.GatherScatterMode.PROMISE_IN_BOUNDS)


def _dense_body(with_prev, with_relu, *refs):
    if with_prev:
        (acc_ref, bias_ref, w_ref, att2_ref,
         g_ref, asrc_ref, adst_ref) = refs
        a0 = acc_ref[0]
        a1 = acc_ref[1]
        den = jnp.maximum(a1[:, ONES:ONES + 1], 1e-16)
        f = jnp.concatenate([a0, a1[:, :ONES]], axis=1) / den + bias_ref[...]
        if with_relu:
            f = jnp.maximum(f, 0.0)
    else:
        (x_ref, w_ref, att2_ref, g_ref, asrc_ref, adst_ref) = refs
        f = x_ref[...]
    h = lax.dot_general(f, w_ref[...], (((1,), (1,)), ((), ())),
                        precision=_HIGH, preferred_element_type=jnp.float32)
    g_ref[0] = h[:, :HALF]
    col = lax.broadcasted_iota(jnp.int32, (h.shape[0], HALF - ONES), 1)
    g_ref[1] = jnp.concatenate(
        [h[:, HALF:], jnp.where(col == 0, 1.0, 0.0)], axis=1)
    ab = lax.dot_general(h, att2_ref[...], (((1,), (0,)), ((), ())),
                         precision=_HIGH, preferred_element_type=jnp.float32)
    asrc_ref[...] = ab[:, 0:1]
    adst_ref[...] = ab[:, 1:2]


def _dense_stage(f_or_acc, w, att_src, att_dst, bias=None, with_relu=False):
    att2 = jnp.stack([att_src, att_dst], axis=1)
    out_shape = [
        jax.ShapeDtypeStruct((NC, N, HALF), jnp.float32),
        jax.ShapeDtypeStruct((N, 1), jnp.float32),
        jax.ShapeDtypeStruct((N, 1), jnp.float32),
    ]
    if bias is None:
        body = functools.partial(_dense_body, False, False)
        args = (f_or_acc, w, att2)
    else:
        body = functools.partial(_dense_body, True, with_relu)
        args = (f_or_acc, bias.reshape(1, HID), w, att2)
    g, asrc, adst = pl.pallas_call(body, out_shape=out_shape)(*args)
    return g, asrc.reshape(N), adst.reshape(N)


MAIN = NBLKW - 2


def _scale_rows(rows, ib, asv, adv, mvec, slot):
    @plsc.parallel_loop(0, BLK // L, unroll=BLK // L)
    def _(j):
        si = ib[slot, 0, pl.ds(j * L, L)]
        di = ib[slot, 1, pl.ds(j * L, L)]
        a_s = plsc.load_gather(asv, [si])
        a_d = plsc.load_gather(adv, [di])
        z = a_s + a_d
        e = jnp.maximum(z, 0.2 * z)
        zb = a_d + mvec
        mb = jnp.maximum(zb, 0.2 * zb)
        ex = jnp.exp(e - mb)
        for r in range(L):
            bv = _bcast_lane(ex, r)
            row = j * L + r
            for k in range(HALF // L):
                sl = (slot, row, pl.ds(k * L, L))
                rows[sl] = rows[sl] * bv


def _edge_body(g_hbm, asrc_hbm, adst_hbm, mvec_hbm, epk_hbm, zero_hbm,
               acc_hbm, asv, adv, ib, scidx, rows, mv, acc_sh,
               isem, gsem, ssem, dsem):
    c = lax.axis_index("c")
    s = lax.axis_index("s")

    pltpu.sync_copy(asrc_hbm, asv)
    pltpu.sync_copy(adst_hbm, adv)
    pltpu.sync_copy(mvec_hbm, mv)

    pltpu.sync_copy(zero_hbm, acc_sh.at[pl.ds(s * RPW, RPW), :])
    plsc.subcore_barrier()

    ghalf = g_hbm.at[c]
    b0 = s * NBLKW
    mvec = mv[...]

    def fetch_idx(b, slot):
        pltpu.async_copy(epk_hbm.at[b0 + b], ib.at[slot], isem.at[slot])

    def wait_idx(b, slot):
        pltpu.make_async_copy(epk_hbm.at[b0 + b], ib.at[slot],
                              isem.at[slot]).wait()

    def start_gather(slot):
        pltpu.async_copy(ghalf.at[ib.at[slot, 0]], rows.at[slot],
                         gsem.at[slot])

    def wait_gather(slot):
        pltpu.make_async_copy(ghalf.at[ib.at[slot, 0]], rows.at[slot],
                              gsem.at[slot]).wait()

    def fetch_didx(b, slot):
        pltpu.async_copy(epk_hbm.at[b0 + b, 1], scidx.at[slot], dsem.at[slot])

    def wait_didx(b, slot):
        pltpu.make_async_copy(epk_hbm.at[b0 + b, 1], scidx.at[slot],
                              dsem.at[slot]).wait()

    def start_scatter(slot):
        pltpu.async_copy(rows.at[slot], acc_sh.at[scidx.at[slot]],
                         ssem.at[slot], add=True)

    def wait_scatter(slot):
        pltpu.make_async_copy(rows.at[slot], acc_sh.at[scidx.at[slot]],
                              ssem.at[slot]).wait()

    for k in range(NSLOT):
        fetch_idx(k, k)
    for k in range(2):
        fetch_didx(k, k)
        wait_idx(k, k)
        start_gather(k)

    @pl.loop(0, MAIN, step=NSLOT)
    def _(p):
        for k in range(NSLOT):
            b = p + k
            wait_gather(k)
            wait_didx(b, k)
            _scale_rows(rows, ib, asv, adv, mvec, k)

            @pl.when(b >= 2)
            def _():
                wait_scatter((k + 2) % NSLOT)
            start_scatter(k)

            @pl.when(b + NSLOT < NBLKW)
            def _():
                fetch_idx(b + NSLOT, k)

            fetch_didx(b + 2, (k + 2) % NSLOT)
            wait_idx(b + 2, (k + 2) % NSLOT)
            start_gather((k + 2) % NSLOT)

    for t in range(2):
        wait_gather(t)
        wait_didx(MAIN + t, t)
        _scale_rows(rows, ib, asv, adv, mvec, t)
        wait_scatter((t + 2) % NSLOT)
        start_scatter(t)
    wait_scatter(0)
    wait_scatter(1)

    plsc.subcore_barrier()
    pltpu.sync_copy(acc_sh.at[pl.ds(s * RPW, RPW), :],
                    acc_hbm.at[c, pl.ds(s * RPW, RPW), :])


_edge_kernel = pl.kernel(
    _edge_body,
    out_type=jax.ShapeDtypeStruct((NC, N, HALF), jnp.float32),
    mesh=plsc.VectorSubcoreMesh(core_axis_name="c", subcore_axis_name="s"),
    scratch_types=[
        pltpu.VMEM((N,), jnp.float32),
        pltpu.VMEM((N,), jnp.float32),
        pltpu.VMEM((NSLOT, 2, BLK), jnp.int32),
        pltpu.VMEM((NSLOT, BLK), jnp.int32),
        pltpu.VMEM((NSLOT, BLK, HALF), jnp.float32),
        pltpu.VMEM((L,), jnp.float32),
        pltpu.VMEM_SHARED((N, HALF), jnp.float32),
        pltpu.SemaphoreType.DMA((NSLOT,)),
        pltpu.SemaphoreType.DMA((NSLOT,)),
        pltpu.SemaphoreType.DMA((NSLOT,)),
        pltpu.SemaphoreType.DMA((NSLOT,)),
    ],
    compiler_params=pltpu.CompilerParams(use_tc_tiling_on_sc=False,
                                         needs_layout_passes=False),
)


def _edge_stage(g, asrc, adst, epk, zero_rows):
    m = jnp.max(asrc)
    mvec = jnp.full((L,), m, dtype=jnp.float32)
    return _edge_kernel(g, asrc, adst, mvec, epk, zero_rows)


def _head_body(acc_ref, bias_ref, batch_ref, wlin_ref, blin_ref,
               out_ref, xmid_ref):
    a0 = acc_ref[0]
    a1 = acc_ref[1]
    den = jnp.maximum(a1[:, ONES:ONES + 1], 1e-16)
    f = jnp.concatenate([a0, a1[:, :ONES]], axis=1) / den
    bt = batch_ref[...]
    gids = lax.broadcasted_iota(jnp.int32, (G, N), 0)
    oh = jnp.where(gids == bt, 1.0, 0.0)
    psum = lax.dot_general(oh, f, (((1,), (0,)), ((), ())),
                           precision=_HIGH, preferred_element_type=jnp.float32)
    cnt = jnp.sum(oh, axis=1, keepdims=True)
    pooled = psum / jnp.maximum(cnt, 1.0) + bias_ref[...]
    xmid_ref[...] = pooled
    out_ref[...] = lax.dot_general(
        pooled, wlin_ref[...], (((1,), (1,)), ((), ())),
        precision=_HIGH, preferred_element_type=jnp.float32) + blin_ref[...]


def _head_stage(acc, b3, batch, wlin, blin):
    out_shape = [
        jax.ShapeDtypeStruct((G, C), jnp.float32),
        jax.ShapeDtypeStruct((G, HID), jnp.float32),
    ]
    return pl.pallas_call(_head_body, out_shape=out_shape)(
        acc, b3.reshape(1, HID), batch.reshape(1, N).astype(jnp.int32),
        wlin, blin.reshape(1, C))


def kernel(x, edge_index, batch, W1, att_src1, att_dst1, b1,
           W2, att_src2, att_dst2, b2, W3, att_src3, att_dst3, b3,
           Wlin, blin):
    epk = jnp.transpose(edge_index.astype(jnp.int32).reshape(2, E // BLK, BLK),
                        (1, 0, 2))
    zero_rows = jnp.zeros((RPW, HALF), jnp.float32)

    g1, as1, ad1 = _dense_stage(x, W1, att_src1, att_dst1)
    acc1 = _edge_stage(g1, as1, ad1, epk, zero_rows)

    g2, as2, ad2 = _dense_stage(acc1, W2, att_src2, att_dst2,
                                bias=b1, with_relu=True)
    acc2 = _edge_stage(g2, as2, ad2, epk, zero_rows)

    g3, as3, ad3 = _dense_stage(acc2, W3, att_src3, att_dst3,
                                bias=b2, with_relu=True)
    acc3 = _edge_stage(g3, as3, ad3, epk, zero_rows)

    return _head_stage(acc3, b3, batch, Wlin, blin)


# speedup vs baseline: 1.7573x; 1.0076x over previous
import functools

import jax
import jax.numpy as jnp
from jax import lax
from jax.experimental import pallas as pl
from jax.experimental.pallas import tpu as pltpu
from jax.experimental.pallas import tpu_sc as plsc

N = 10000
E = 320000
D = 128
HID = 128
G = 64
C = 16

HALF = 80
ONES = D - HALF
NC = 2
NS = 16
BLK = 80
NBLKW = E // NS // BLK
L = 16
RPW = N // NS
NSLOT = 4

_HIGH = lax.Precision.HIGHEST


def _bcast_lane(vec, r):
    idx = jnp.full((L, 1), r, dtype=jnp.int32)
    return lax.gather(
        vec, idx,
        lax.GatherDimensionNumbers(
            offset_dims=(), collapsed_slice_dims=(0,), start_index_map=(0,)),
        slice_sizes=(1,),
        mode=lax.GatherScatterMode.PROMISE_IN_BOUNDS)


def _dense_body(with_prev, with_relu, *refs):
    if with_prev:
        (acc_ref, bias_ref, w_ref, att2_ref,
         g_ref, asrc_ref, adst_ref, mvec_ref) = refs
        a0 = acc_ref[0]
        a1 = acc_ref[1]
        den = jnp.maximum(a1[:, ONES:ONES + 1], 1e-16)
        f = jnp.concatenate([a0, a1[:, :ONES]], axis=1) / den + bias_ref[...]
        if with_relu:
            f = jnp.maximum(f, 0.0)
    else:
        (x_ref, w_ref, att2_ref, g_ref, asrc_ref, adst_ref, mvec_ref) = refs
        f = x_ref[...]
    h = lax.dot_general(f, w_ref[...], (((1,), (1,)), ((), ())),
                        precision=_HIGH, preferred_element_type=jnp.float32)
    g_ref[0] = h[:, :HALF]
    col = lax.broadcasted_iota(jnp.int32, (h.shape[0], HALF - ONES), 1)
    g_ref[1] = jnp.concatenate(
        [h[:, HALF:], jnp.where(col == 0, 1.0, 0.0)], axis=1)
    ab = lax.dot_general(h, att2_ref[...], (((1,), (0,)), ((), ())),
                         precision=_HIGH, preferred_element_type=jnp.float32)
    asrc_ref[...] = ab[:, 0:1]
    adst_ref[...] = ab[:, 1:2]
    mvec_ref[...] = jnp.full((1, L), jnp.max(ab[:, 0]), dtype=jnp.float32)


def _dense_stage(f_or_acc, w, att_src, att_dst, bias=None, with_relu=False):
    att2 = jnp.stack([att_src, att_dst], axis=1)
    out_shape = [
        jax.ShapeDtypeStruct((NC, N, HALF), jnp.float32),
        jax.ShapeDtypeStruct((N, 1), jnp.float32),
        jax.ShapeDtypeStruct((N, 1), jnp.float32),
        jax.ShapeDtypeStruct((1, L), jnp.float32),
    ]
    if bias is None:
        body = functools.partial(_dense_body, False, False)
        args = (f_or_acc, w, att2)
    else:
        body = functools.partial(_dense_body, True, with_relu)
        args = (f_or_acc, bias.reshape(1, HID), w, att2)
    g, asrc, adst, mvec = pl.pallas_call(body, out_shape=out_shape)(*args)
    return g, asrc.reshape(N), adst.reshape(N), mvec.reshape(L)


MAIN = NBLKW - 2


def _scale_rows(rows, ib, asv, adv, mvec, slot):
    @plsc.parallel_loop(0, BLK // L, unroll=BLK // L)
    def _(j):
        si = ib[slot, 0, pl.ds(j * L, L)]
        di = ib[slot, 1, pl.ds(j * L, L)]
        a_s = plsc.load_gather(asv, [si])
        a_d = plsc.load_gather(adv, [di])
        z = a_s + a_d
        e = jnp.maximum(z, 0.2 * z)
        zb = a_d + mvec
        mb = jnp.maximum(zb, 0.2 * zb)
        ex = jnp.exp(e - mb)
        for r in range(L):
            bv = _bcast_lane(ex, r)
            row = j * L + r
            for k in range(HALF // L):
                sl = (slot, row, pl.ds(k * L, L))
                rows[sl] = rows[sl] * bv


def _edge_body(g_hbm, asrc_hbm, adst_hbm, mvec_hbm, epk_hbm, zero_hbm,
               acc_hbm, asv, adv, ib, scidx, rows, mv, acc_sh,
               isem, gsem, ssem, dsem):
    c = lax.axis_index("c")
    s = lax.axis_index("s")

    pltpu.sync_copy(asrc_hbm, asv)
    pltpu.sync_copy(adst_hbm, adv)
    pltpu.sync_copy(mvec_hbm, mv)

    pltpu.sync_copy(zero_hbm, acc_sh.at[pl.ds(s * RPW, RPW), :])
    plsc.subcore_barrier()

    ghalf = g_hbm.at[c]
    b0 = s * NBLKW
    mvec = mv[...]

    def fetch_idx(b, slot):
        pltpu.async_copy(epk_hbm.at[b0 + b], ib.at[slot], isem.at[slot])

    def wait_idx(b, slot):
        pltpu.make_async_copy(epk_hbm.at[b0 + b], ib.at[slot],
                              isem.at[slot]).wait()

    def start_gather(slot):
        pltpu.async_copy(ghalf.at[ib.at[slot, 0]], rows.at[slot],
                         gsem.at[slot])

    def wait_gather(slot):
        pltpu.make_async_copy(ghalf.at[ib.at[slot, 0]], rows.at[slot],
                              gsem.at[slot]).wait()

    def fetch_didx(b, slot):
        pltpu.async_copy(epk_hbm.at[b0 + b, 1], scidx.at[slot], dsem.at[slot])

    def wait_didx(b, slot):
        pltpu.make_async_copy(epk_hbm.at[b0 + b, 1], scidx.at[slot],
                              dsem.at[slot]).wait()

    def start_scatter(slot):
        pltpu.async_copy(rows.at[slot], acc_sh.at[scidx.at[slot]],
                         ssem.at[slot], add=True)

    def wait_scatter(slot):
        pltpu.make_async_copy(rows.at[slot], acc_sh.at[scidx.at[slot]],
                              ssem.at[slot]).wait()

    for k in range(NSLOT):
        fetch_idx(k, k)
    for k in range(2):
        fetch_didx(k, k)
        wait_idx(k, k)
        start_gather(k)

    @pl.loop(0, MAIN, step=NSLOT)
    def _(p):
        for k in range(NSLOT):
            b = p + k
            wait_gather(k)
            wait_didx(b, k)
            _scale_rows(rows, ib, asv, adv, mvec, k)

            @pl.when(b >= 2)
            def _():
                wait_scatter((k + 2) % NSLOT)
            start_scatter(k)

            @pl.when(b + NSLOT < NBLKW)
            def _():
                fetch_idx(b + NSLOT, k)

            fetch_didx(b + 2, (k + 2) % NSLOT)
            wait_idx(b + 2, (k + 2) % NSLOT)
            start_gather((k + 2) % NSLOT)

    for t in range(2):
        wait_gather(t)
        wait_didx(MAIN + t, t)
        _scale_rows(rows, ib, asv, adv, mvec, t)
        wait_scatter((t + 2) % NSLOT)
        start_scatter(t)
    wait_scatter(0)
    wait_scatter(1)

    plsc.subcore_barrier()
    pltpu.sync_copy(acc_sh.at[pl.ds(s * RPW, RPW), :],
                    acc_hbm.at[c, pl.ds(s * RPW, RPW), :])


_edge_kernel = pl.kernel(
    _edge_body,
    out_type=jax.ShapeDtypeStruct((NC, N, HALF), jnp.float32),
    mesh=plsc.VectorSubcoreMesh(core_axis_name="c", subcore_axis_name="s"),
    scratch_types=[
        pltpu.VMEM((N,), jnp.float32),
        pltpu.VMEM((N,), jnp.float32),
        pltpu.VMEM((NSLOT, 2, BLK), jnp.int32),
        pltpu.VMEM((NSLOT, BLK), jnp.int32),
        pltpu.VMEM((NSLOT, BLK, HALF), jnp.float32),
        pltpu.VMEM((L,), jnp.float32),
        pltpu.VMEM_SHARED((N, HALF), jnp.float32),
        pltpu.SemaphoreType.DMA((NSLOT,)),
        pltpu.SemaphoreType.DMA((NSLOT,)),
        pltpu.SemaphoreType.DMA((NSLOT,)),
        pltpu.SemaphoreType.DMA((NSLOT,)),
    ],
    compiler_params=pltpu.CompilerParams(use_tc_tiling_on_sc=False,
                                         needs_layout_passes=False),
)


def _edge_stage(g, asrc, adst, mvec, epk, zero_rows):
    return _edge_kernel(g, asrc, adst, mvec, epk, zero_rows)


def _head_body(acc_ref, bias_ref, batch_ref, wlin_ref, blin_ref,
               out_ref, xmid_ref):
    a0 = acc_ref[0]
    a1 = acc_ref[1]
    den = jnp.maximum(a1[:, ONES:ONES + 1], 1e-16)
    f = jnp.concatenate([a0, a1[:, :ONES]], axis=1) / den
    bt = batch_ref[...]
    gids = lax.broadcasted_iota(jnp.int32, (G, N), 0)
    oh = jnp.where(gids == bt, 1.0, 0.0)
    psum = lax.dot_general(oh, f, (((1,), (0,)), ((), ())),
                           precision=_HIGH, preferred_element_type=jnp.float32)
    cnt = jnp.sum(oh, axis=1, keepdims=True)
    pooled = psum / jnp.maximum(cnt, 1.0) + bias_ref[...]
    xmid_ref[...] = pooled
    out_ref[...] = lax.dot_general(
        pooled, wlin_ref[...], (((1,), (1,)), ((), ())),
        precision=_HIGH, preferred_element_type=jnp.float32) + blin_ref[...]


def _head_stage(acc, b3, batch, wlin, blin):
    out_shape = [
        jax.ShapeDtypeStruct((G, C), jnp.float32),
        jax.ShapeDtypeStruct((G, HID), jnp.float32),
    ]
    return pl.pallas_call(_head_body, out_shape=out_shape)(
        acc, b3.reshape(1, HID), batch.reshape(1, N).astype(jnp.int32),
        wlin, blin.reshape(1, C))


def kernel(x, edge_index, batch, W1, att_src1, att_dst1, b1,
           W2, att_src2, att_dst2, b2, W3, att_src3, att_dst3, b3,
           Wlin, blin):
    epk = jnp.transpose(edge_index.astype(jnp.int32).reshape(2, E // BLK, BLK),
                        (1, 0, 2))
    zero_rows = jnp.zeros((RPW, HALF), jnp.float32)

    g1, as1, ad1, mv1 = _dense_stage(x, W1, att_src1, att_dst1)
    acc1 = _edge_stage(g1, as1, ad1, mv1, epk, zero_rows)

    g2, as2, ad2, mv2 = _dense_stage(acc1, W2, att_src2, att_dst2,
                                     bias=b1, with_relu=True)
    acc2 = _edge_stage(g2, as2, ad2, mv2, epk, zero_rows)

    g3, as3, ad3, mv3 = _dense_stage(acc2, W3, att_src3, att_dst3,
                                     bias=b2, with_relu=True)
    acc3 = _edge_stage(g3, as3, ad3, mv3, epk, zero_rows)

    return _head_stage(acc3, b3, batch, Wlin, blin)
